# baseline (speedup 1.0000x reference)
import functools

import jax
import jax.numpy as jnp
from jax.experimental import pallas as pl
from jax.experimental.pallas import tpu as pltpu

_NBVEC = 8


def _body(x_ref, score_ref, vec_ref, sim_ref):
    bb = x_ref.shape[0]
    n = x_ref.shape[2]
    dk = (((1,), (0,)), ((), ()))
    iota = jax.lax.broadcasted_iota(jnp.int32, (1, n), 1)
    x2s, xts, n2s, scores = [], [], [], []
    for b in range(bb):
        x2 = x_ref[b]
        x2s.append(x2)
        xts.append(x2.T)
        n2s.append(jnp.sum(x2 * x2, axis=0, keepdims=True))
        scores.append(score_ref[b])
    def one_iter(i, carry):
        new_scores = []
        for b in range(bb):
            x2, xt, n2, score = x2s[b], xts[b], n2s[b], carry[b]
            m = jnp.max(score)
            idx = jnp.min(jnp.where(score == m, iota, n))
            onehot = (iota == idx).astype(jnp.float32)
            raw = jax.lax.dot_general(onehot, xt, dk,
                                      preferred_element_type=jnp.float32)
            r2 = jnp.sum(raw * raw)
            dot = jax.lax.dot_general(raw, x2, dk,
                                      preferred_element_type=jnp.float32)
            d2 = jnp.maximum(n2 - 2.0 * dot + r2, 0.0)
            d2 = jnp.where(iota == idx, 0.0, d2)
            sim = jnp.exp(-jnp.sqrt(d2) * (1.0 / 20.0))
            ssum = jnp.sum(sim)
            wsum = jax.lax.dot_general(sim, xt, dk,
                                       preferred_element_type=jnp.float32)
            vec_ref[b, pl.ds(i, 1), :] = wsum / ssum
            sim_ref[b, pl.ds(i, 1), :] = sim
            new_scores.append((1.0 - sim) * score)
        return tuple(new_scores)

    jax.lax.fori_loop(0, _NBVEC, one_iter, tuple(scores))


def kernel(x, applyUMAP):
    del applyUMAP
    B, C, H, W = x.shape
    n = H * W
    x3 = x.reshape(B, C, n)
    score0 = jax.random.uniform(jax.random.key(1), (B, n), dtype=x.dtype)
    score0 = score0.reshape(B, 1, n)
    BB = 2
    vecs, sims = pl.pallas_call(
        _body,
        grid=(B // BB,),
        in_specs=[
            pl.BlockSpec((BB, C, n), lambda b: (b, 0, 0)),
            pl.BlockSpec((BB, 1, n), lambda b: (b, 0, 0)),
        ],
        out_specs=[
            pl.BlockSpec((BB, _NBVEC, C), lambda b: (b, 0, 0)),
            pl.BlockSpec((BB, _NBVEC, n), lambda b: (b, 0, 0)),
        ],
        out_shape=[
            jax.ShapeDtypeStruct((B, _NBVEC, C), x.dtype),
            jax.ShapeDtypeStruct((B, _NBVEC, n), x.dtype),
        ],
    )(x3, score0)
    selectedPos = jnp.zeros((B, 1, H, W), dtype=x.dtype)
    return (vecs, sims.reshape(B, _NBVEC, H, W), selectedPos)


# speedup vs baseline: 1.1529x; 1.1529x over previous
import functools

import jax
import jax.numpy as jnp
from jax.experimental import pallas as pl
from jax.experimental.pallas import tpu as pltpu

_NBVEC = 8


def _body(x_ref, score_ref, vec_ref, sim_ref):
    bb = x_ref.shape[0]
    n = x_ref.shape[2]
    dn = (((1,), (1,)), ((), ()))
    iota = jax.lax.broadcasted_iota(jnp.int32, (1, n), 1)
    x2s, n2s, scores = [], [], []
    for b in range(bb):
        x2 = x_ref[b]
        x2s.append(x2)
        n2s.append(jnp.sum(x2 * x2, axis=0, keepdims=True))
        scores.append(score_ref[b])
    def one_iter(i, carry):
        new_scores = []
        for b in range(bb):
            x2, n2, score = x2s[b], n2s[b], carry[b]
            m = jnp.max(score)
            idx = jnp.min(jnp.where(score == m, iota, n))
            onehot = (iota == idx).astype(jnp.float32)
            raw = jax.lax.dot_general(x2, onehot, dn,
                                      preferred_element_type=jnp.float32)
            r2 = jnp.sum(raw * raw)
            dot = jax.lax.dot_general(
                raw.T, x2, (((1,), (0,)), ((), ())),
                preferred_element_type=jnp.float32)
            d2 = jnp.maximum(n2 - 2.0 * dot + r2, 0.0)
            d2 = jnp.where(iota == idx, 0.0, d2)
            sim = jnp.exp(-jnp.sqrt(d2) * (1.0 / 20.0))
            ssum = jnp.sum(sim)
            wsum = jax.lax.dot_general(x2, sim, dn,
                                       preferred_element_type=jnp.float32)
            vec_ref[b, pl.ds(i, 1), :] = (wsum[:, 0] / ssum)[None, :]
            sim_ref[b, pl.ds(i, 1), :] = sim
            new_scores.append((1.0 - sim) * score)
        return tuple(new_scores)

    jax.lax.fori_loop(0, _NBVEC, one_iter, tuple(scores))


def kernel(x, applyUMAP):
    del applyUMAP
    B, C, H, W = x.shape
    n = H * W
    x3 = x.reshape(B, C, n)
    score0 = jax.random.uniform(jax.random.key(1), (B, n), dtype=x.dtype)
    score0 = score0.reshape(B, 1, n)
    BB = 2
    vecs, sims = pl.pallas_call(
        _body,
        grid=(B // BB,),
        in_specs=[
            pl.BlockSpec((BB, C, n), lambda b: (b, 0, 0)),
            pl.BlockSpec((BB, 1, n), lambda b: (b, 0, 0)),
        ],
        out_specs=[
            pl.BlockSpec((BB, _NBVEC, C), lambda b: (b, 0, 0)),
            pl.BlockSpec((BB, _NBVEC, n), lambda b: (b, 0, 0)),
        ],
        out_shape=[
            jax.ShapeDtypeStruct((B, _NBVEC, C), x.dtype),
            jax.ShapeDtypeStruct((B, _NBVEC, n), x.dtype),
        ],
    )(x3, score0)
    selectedPos = jnp.zeros((B, 1, H, W), dtype=x.dtype)
    return (vecs, sims.reshape(B, _NBVEC, H, W), selectedPos)


# speedup vs baseline: 2.8913x; 2.5079x over previous
import jax
import jax.numpy as jnp
from jax.experimental import pallas as pl

_NBVEC = 8


def _body(x_ref, score_ref, vec_ref, sim_ref):
    x4 = x_ref[0]
    score = score_ref[0]
    C, H, W = x4.shape
    ir = jax.lax.broadcasted_iota(jnp.int32, (H, W), 0)
    ic = jax.lax.broadcasted_iota(jnp.int32, (H, W), 1)
    pidx = ir * W + ic
    lane = jax.lax.broadcasted_iota(jnp.int32, (1, W), 1)
    n2 = jnp.sum(x4 * x4, axis=0)
    for i in range(_NBVEC):
        m = jnp.max(score)
        idx = jnp.min(jnp.where(score == m, pidx, H * W))
        r = idx // W
        c = idx % W
        rowblk = x_ref[0, :, pl.ds(r, 1), :][:, 0, :]
        onehot = (lane == c).astype(jnp.float32)
        raw = jnp.sum(rowblk * onehot, axis=1, keepdims=True)
        r2 = jnp.sum(raw * raw)
        dot = jnp.sum(x4 * raw[:, :, None], axis=0)
        d2 = jnp.maximum(n2 - 2.0 * dot + r2, 0.0)
        d2 = jnp.where(pidx == idx, 0.0, d2)
        sim = jnp.exp(-jnp.sqrt(d2) * (1.0 / 20.0))
        ssum = jnp.sum(sim)
        wsum = jnp.sum(x4 * sim[None, :, :], axis=(1, 2))
        vec_ref[0, i, :] = wsum / ssum
        sim_ref[0, i, :, :] = sim
        score = (1.0 - sim) * score


def kernel(x, applyUMAP):
    del applyUMAP
    B, C, H, W = x.shape
    n = H * W
    score0 = jax.random.uniform(jax.random.key(1), (B, n), dtype=x.dtype)
    score0 = score0.reshape(B, H, W)
    vecs, sims = pl.pallas_call(
        _body,
        grid=(B,),
        in_specs=[
            pl.BlockSpec((1, C, H, W), lambda b: (b, 0, 0, 0)),
            pl.BlockSpec((1, H, W), lambda b: (b, 0, 0)),
        ],
        out_specs=[
            pl.BlockSpec((1, _NBVEC, C), lambda b: (b, 0, 0)),
            pl.BlockSpec((1, _NBVEC, H, W), lambda b: (b, 0, 0, 0)),
        ],
        out_shape=[
            jax.ShapeDtypeStruct((B, _NBVEC, C), x.dtype),
            jax.ShapeDtypeStruct((B, _NBVEC, H, W), x.dtype),
        ],
    )(x, score0)
    selectedPos = jnp.zeros((B, 1, H, W), dtype=x.dtype)
    return (vecs, sims, selectedPos)


# speedup vs baseline: 2.9435x; 1.0181x over previous
import jax
import jax.numpy as jnp
from jax.experimental import pallas as pl

_NBVEC = 8


def _body(x_ref, score_ref, vec_ref, sim_ref):
    x4 = x_ref[0]
    score = score_ref[0]
    C, H, W = x4.shape
    ir = jax.lax.broadcasted_iota(jnp.int32, (H, W), 0)
    ic = jax.lax.broadcasted_iota(jnp.int32, (H, W), 1)
    pidx = ir * W + ic
    lane = jax.lax.broadcasted_iota(jnp.int32, (1, W), 1)
    n2 = x4[0] * x4[0]
    for cc in range(1, C):
        n2 = n2 + x4[cc] * x4[cc]
    for i in range(_NBVEC):
        m = jnp.max(score)
        idx = jnp.min(jnp.where(score == m, pidx, H * W))
        r = idx // W
        c = idx % W
        rowblk = x_ref[0, :, pl.ds(r, 1), :][:, 0, :]
        onehot = (lane == c).astype(jnp.float32)
        raw = jnp.sum(rowblk * onehot, axis=1, keepdims=True)
        r2 = jnp.sum(raw * raw)
        dot = jnp.sum(x4 * raw[:, :, None], axis=0)
        d2 = jnp.maximum(n2 - 2.0 * dot + r2, 0.0)
        d2 = jnp.where(pidx == idx, 0.0, d2)
        sim = jnp.exp(-jnp.sqrt(d2) * (1.0 / 20.0))
        ssum = jnp.sum(sim)
        wsum = jnp.sum(x4 * sim[None, :, :], axis=(1, 2))
        vec_ref[0, i, :] = wsum / ssum
        sim_ref[0, i, :, :] = sim
        score = (1.0 - sim) * score


def kernel(x, applyUMAP):
    del applyUMAP
    B, C, H, W = x.shape
    n = H * W
    score0 = jax.random.uniform(jax.random.key(1), (B, n), dtype=x.dtype)
    score0 = score0.reshape(B, H, W)
    vecs, sims = pl.pallas_call(
        _body,
        grid=(B,),
        in_specs=[
            pl.BlockSpec((1, C, H, W), lambda b: (b, 0, 0, 0)),
            pl.BlockSpec((1, H, W), lambda b: (b, 0, 0)),
        ],
        out_specs=[
            pl.BlockSpec((1, _NBVEC, C), lambda b: (b, 0, 0)),
            pl.BlockSpec((1, _NBVEC, H, W), lambda b: (b, 0, 0, 0)),
        ],
        out_shape=[
            jax.ShapeDtypeStruct((B, _NBVEC, C), x.dtype),
            jax.ShapeDtypeStruct((B, _NBVEC, H, W), x.dtype),
        ],
    )(x, score0)
    selectedPos = jnp.zeros((B, 1, H, W), dtype=x.dtype)
    return (vecs, sims, selectedPos)
